# baseline (speedup 1.0000x reference)
import functools

import numpy as np

import jax
import jax.numpy as jnp
from jax.experimental import pallas as pl
from jax.experimental.pallas import tpu as pltpu
from jax.experimental.pallas import tpu_sc as plsc

_P = 16
_B = 16
_H = 512
_W = 512
_HP = _H // _P
_WP = _W // _P
_NP = _HP * _WP
_TBL = 1040
_CH = 32
_NCH = 256 // _CH

_U32 = np.uint64(0xFFFFFFFF)
_ROT = ((13, 15, 26, 6), (17, 29, 16, 24))


def _threefry2x32(k0, k1, x0, x1):
    ks0 = np.uint64(k0) & _U32
    ks1 = np.uint64(k1) & _U32
    ks2 = ks0 ^ ks1 ^ np.uint64(0x1BD11BDA)
    x0 = (np.asarray(x0, np.uint64) + ks0) & _U32
    x1 = (np.asarray(x1, np.uint64) + ks1) & _U32
    sched = ((ks1, ks2), (ks2, ks0), (ks0, ks1), (ks1, ks2), (ks2, ks0))
    for r in range(5):
        for d in _ROT[r % 2]:
            x0 = (x0 + x1) & _U32
            x1 = (((x1 << np.uint64(d)) | (x1 >> np.uint64(32 - d))) & _U32) ^ x0
        a, b = sched[r]
        x0 = (x0 + a) & _U32
        x1 = (x1 + b + np.uint64(r + 1)) & _U32
    return x0, x1


def _build_draw_table():
    tbl = np.zeros((_B, _TBL), np.int32)
    spans = np.maximum(np.arange(_TBL, dtype=np.uint64), 1)
    for i in range(_B):
        ki = _threefry2x32(0, 42, np.uint64(0), np.uint64(i))
        y0, y1 = _threefry2x32(ki[0], ki[1], np.array([0, 0], np.uint64),
                               np.array([0, 1], np.uint64))
        sub1, sub2 = (y0[0], y1[0]), (y0[1], y1[1])
        hi0, hi1 = _threefry2x32(sub1[0], sub1[1], np.uint64(0), np.uint64(0))
        lo0, lo1 = _threefry2x32(sub2[0], sub2[1], np.uint64(0), np.uint64(0))
        higher = np.uint64(hi0 ^ hi1)
        lower = np.uint64(lo0 ^ lo1)
        mult = (np.uint64(2 ** 16) % spans)
        mult = (mult * mult) % spans
        off = ((higher % spans) * mult + (lower % spans)) % spans
        tbl[i] = off.astype(np.int32)
    return tbl


_DRAW_TABLE = _build_draw_table()

_sc_mesh = plsc.VectorSubcoreMesh(core_axis_name="c", subcore_axis_name="s")


@functools.partial(
    pl.kernel,
    mesh=_sc_mesh,
    compiler_params=pltpu.CompilerParams(needs_layout_passes=False),
    out_type=[
        jax.ShapeDtypeStruct((_B // 2, 16), jnp.int32),
        jax.ShapeDtypeStruct((_B // 2, 16), jnp.int32),
    ],
    scratch_types=[
        pltpu.VMEM((_CH, _W), jnp.float32),
        pltpu.VMEM((_CH, _W), jnp.float32),
        pltpu.VMEM((16, _W), jnp.float32),
        pltpu.VMEM((16, _WP), jnp.float32),
        pltpu.VMEM((_HP, _WP), jnp.float32),
        pltpu.VMEM((_TBL,), jnp.int32),
        pltpu.VMEM((16,), jnp.int32),
        pltpu.VMEM_SHARED((16, 16, _WP), jnp.float32),
        pltpu.SemaphoreType.DMA,
        pltpu.SemaphoreType.DMA,
    ],
)
def _sc_kernel(mask_hbm, tbl_hbm, out0_hbm, out1_hbm, buf0, buf1, rows_v,
               psum_v, res_v, tbl_v, out_v, shared, sem0, sem1):
    c = jax.lax.axis_index("c")
    s = jax.lax.axis_index("s")
    b = 8 * c + s // 2
    half = s % 2
    base = 256 * half
    bufs = (buf0, buf1)
    sems = (sem0, sem1)
    lanes = jax.lax.iota(jnp.int32, 16)

    def cp(i, j):
        return pltpu.make_async_copy(
            mask_hbm.at[b, 0, pl.ds(base + _CH * i, _CH)], bufs[j], sems[j])

    cp(0, 0).start()
    cp(1, 1).start()

    def group(g, carry):
        for j in range(2):
            i = 2 * g + j
            cp(i, j).wait()

            def prbody(pr, cc):
                r0 = 16 * pr
                for v in range(_W // 16):
                    sl = pl.ds(16 * v, 16)
                    vals = [bufs[j][r0 + k, sl] for k in range(16)]
                    while len(vals) > 1:
                        vals = [vals[m] + vals[m + 1]
                                for m in range(0, len(vals), 2)]
                    rows_v[(_CH // 16) * i + pr, sl] = vals[0]
                return cc

            jax.lax.fori_loop(0, _CH // 16, prbody, 0)

            @pl.when(g < _NCH // 2 - 1)
            def _():
                cp(i + 2, j).start()
        return carry

    jax.lax.fori_loop(0, _NCH // 2, group, 0)

    def wpool(row, cc):
        ridx = jnp.full((16,), row, jnp.int32)
        for h2 in range(2):
            idx0 = 16 * lanes + 256 * h2
            vals = [plsc.load_gather(rows_v, [ridx, idx0 + l])
                    for l in range(16)]
            while len(vals) > 1:
                vals = [vals[m] + vals[m + 1] for m in range(0, len(vals), 2)]
            psum_v[row, pl.ds(16 * h2, 16)] = vals[0]
        return cc

    jax.lax.fori_loop(0, 16, wpool, 0)

    pltpu.sync_copy(psum_v, shared.at[s])
    plsc.subcore_barrier()

    @pl.when(half == 0)
    def _select():
        pltpu.sync_copy(shared.at[s], res_v.at[pl.ds(0, 16)])
        pltpu.sync_copy(shared.at[s + 1], res_v.at[pl.ds(16, 16)])
        pltpu.sync_copy(tbl_hbm.at[b], tbl_v)

        thresh = jnp.full((16,), float(_P * _P), jnp.float32)
        one = jnp.full((16,), 1, jnp.int32)
        zero = jnp.full((16,), 0, jnp.int32)

        def count(r, n):
            for p in range(2):
                v = res_v[r, pl.ds(p * 16, 16)]
                n = n + jnp.sum(jnp.where(v < thresh, one, zero))
            return n

        n = jax.lax.fori_loop(0, _HP, count, jnp.int32(0))

        jv = plsc.load_gather(tbl_v, [jnp.full((16,), n, jnp.int32)])
        t = jnp.max(jv) + 1

        def select(r, carry):
            run, flat = carry
            for p in range(2):
                v = res_v[r, pl.ds(p * 16, 16)]
                mi = jnp.where(v < thresh, one, zero)
                cs = jax.lax.cumsum(mi)
                cnt = jnp.sum(mi)
                sel = jnp.where(mi == one, cs, zero) == (t - run)
                pos = jnp.min(jnp.where(sel, lanes, 16))
                hit = (run < t) & (t <= run + cnt)
                flat = jnp.where(hit, r * 32 + p * 16 + pos, flat)
                run = run + cnt
            return run, flat

        _, flat = jax.lax.fori_loop(0, _HP, select,
                                    (jnp.int32(0), jnp.int32(0)))

        h = (flat >> 5) << 4
        w = (flat & 31) << 4
        out_v[...] = jnp.where(lanes == 0, h, jnp.where(lanes == 1, w, 0))

        @pl.when(c == 0)
        def _():
            pltpu.sync_copy(out_v, out0_hbm.at[s // 2])

        @pl.when(c == 1)
        def _():
            pltpu.sync_copy(out_v, out1_hbm.at[s // 2])


@jax.jit
def kernel(mask):
    out0, out1 = _sc_kernel(mask, _DRAW_TABLE)
    out = jnp.concatenate([out0, out1], axis=0)
    return out[:, 0], out[:, 1]


# speedup vs baseline: 1.2071x; 1.2071x over previous
import functools

import numpy as np

import jax
import jax.numpy as jnp
from jax.experimental import pallas as pl
from jax.experimental.pallas import tpu as pltpu
from jax.experimental.pallas import tpu_sc as plsc

_P = 16
_B = 16
_H = 512
_W = 512
_HP = _H // _P
_WP = _W // _P
_NP = _HP * _WP
_TBL = 1152
_BTC = 8
_BSC = _B - _BTC
_CH = 32
_QR = 128
_NCH = _QR // _CH

_U32 = np.uint64(0xFFFFFFFF)
_ROT = ((13, 15, 26, 6), (17, 29, 16, 24))


def _threefry2x32(k0, k1, x0, x1):
    ks0 = np.uint64(k0) & _U32
    ks1 = np.uint64(k1) & _U32
    ks2 = ks0 ^ ks1 ^ np.uint64(0x1BD11BDA)
    x0 = (np.asarray(x0, np.uint64) + ks0) & _U32
    x1 = (np.asarray(x1, np.uint64) + ks1) & _U32
    sched = ((ks1, ks2), (ks2, ks0), (ks0, ks1), (ks1, ks2), (ks2, ks0))
    for r in range(5):
        for d in _ROT[r % 2]:
            x0 = (x0 + x1) & _U32
            x1 = (((x1 << np.uint64(d)) | (x1 >> np.uint64(32 - d))) & _U32) ^ x0
        a, b = sched[r]
        x0 = (x0 + a) & _U32
        x1 = (x1 + b + np.uint64(r + 1)) & _U32
    return x0, x1


def _build_draw_table():
    tbl = np.zeros((_B, _TBL), np.int32)
    spans = np.maximum(np.arange(_TBL, dtype=np.uint64), 1)
    for i in range(_B):
        ki = _threefry2x32(0, 42, np.uint64(0), np.uint64(i))
        y0, y1 = _threefry2x32(ki[0], ki[1], np.array([0, 0], np.uint64),
                               np.array([0, 1], np.uint64))
        sub1, sub2 = (y0[0], y1[0]), (y0[1], y1[1])
        hi0, hi1 = _threefry2x32(sub1[0], sub1[1], np.uint64(0), np.uint64(0))
        lo0, lo1 = _threefry2x32(sub2[0], sub2[1], np.uint64(0), np.uint64(0))
        higher = np.uint64(hi0 ^ hi1)
        lower = np.uint64(lo0 ^ lo1)
        mult = (np.uint64(2 ** 16) % spans)
        mult = (mult * mult) % spans
        off = ((higher % spans) * mult + (lower % spans)) % spans
        tbl[i] = off.astype(np.int32)
    return tbl


_DRAW_TABLE = _build_draw_table()
_DRAW_TABLE_3D = _DRAW_TABLE.reshape(_B, 1, _TBL)


def _tc_kernel(mask_ref, tbl_ref, outh_ref, outw_ref):
    x = mask_ref[0, 0]

    gi = jax.lax.broadcasted_iota(jnp.int32, (_HP, _H), 0)
    ci = jax.lax.broadcasted_iota(jnp.int32, (_HP, _H), 1)
    rowpool = (ci // _P == gi).astype(jnp.float32)
    cj = jax.lax.broadcasted_iota(jnp.int32, (_W, _WP), 0)
    gj = jax.lax.broadcasted_iota(jnp.int32, (_W, _WP), 1)
    colpool = (cj // _P == gj).astype(jnp.float32)

    hp = jax.lax.Precision.HIGHEST
    a = jnp.dot(rowpool, x, precision=hp)
    res = jnp.dot(a, colpool, precision=hp)

    cond = (res < float(_P * _P)).astype(jnp.float32)

    rk = jax.lax.broadcasted_iota(jnp.int32, (_WP, _WP), 0)
    ck = jax.lax.broadcasted_iota(jnp.int32, (_WP, _WP), 1)
    upper = (rk <= ck).astype(jnp.float32)
    lower = (ck < rk).astype(jnp.float32)
    within = jnp.dot(cond, upper, precision=hp)
    rowtot = within[:, _WP - 1 :]
    prefix = jnp.dot(lower, rowtot, precision=hp)
    csum = prefix + within

    n = (prefix[_HP - 1, 0] + rowtot[_HP - 1, 0]).astype(jnp.int32)

    trow = tbl_ref[0]
    lane = jax.lax.broadcasted_iota(jnp.int32, (1, _TBL), 1)
    j = jnp.sum(jnp.where(lane == n, trow, 0))

    target = (j + 1).astype(jnp.float32)
    fr = jax.lax.broadcasted_iota(jnp.int32, (_HP, _WP), 0)
    fc = jax.lax.broadcasted_iota(jnp.int32, (_HP, _WP), 1)
    flatidx = fr * _WP + fc
    flat = jnp.min(jnp.where(csum >= target, flatidx, _NP * 4))
    flat = jnp.where(n == 0, 0, flat)

    outh_ref[0, 0, :] = jnp.full((128,), _P * (flat // _WP), jnp.int32)
    outw_ref[0, 0, :] = jnp.full((128,), _P * (flat % _WP), jnp.int32)


_sc_mesh = plsc.VectorSubcoreMesh(core_axis_name="c", subcore_axis_name="s")


@functools.partial(
    pl.kernel,
    mesh=_sc_mesh,
    compiler_params=pltpu.CompilerParams(needs_layout_passes=False),
    out_type=[
        jax.ShapeDtypeStruct((_BSC // 2, 16), jnp.int32),
        jax.ShapeDtypeStruct((_BSC // 2, 16), jnp.int32),
    ],
    scratch_types=[
        pltpu.VMEM((_CH, _W), jnp.float32),
        pltpu.VMEM((_CH, _W), jnp.float32),
        pltpu.VMEM((_QR // 16, _W), jnp.float32),
        pltpu.VMEM((_QR // 16, _WP), jnp.float32),
        pltpu.VMEM((_HP, _WP), jnp.float32),
        pltpu.VMEM((_TBL,), jnp.int32),
        pltpu.VMEM((16,), jnp.int32),
        pltpu.VMEM_SHARED((16, _QR // 16, _WP), jnp.float32),
        pltpu.SemaphoreType.DMA,
        pltpu.SemaphoreType.DMA,
    ],
)
def _sc_kernel(mask_hbm, tbl_hbm, out0_hbm, out1_hbm, buf0, buf1, rows_v,
               psum_v, res_v, tbl_v, out_v, shared, sem0, sem1):
    c = jax.lax.axis_index("c")
    s = jax.lax.axis_index("s")
    b = _BTC + (_BSC // 2) * c + s // 4
    q = s % 4
    base = _QR * q
    bufs = (buf0, buf1)
    sems = (sem0, sem1)
    lanes = jax.lax.iota(jnp.int32, 16)

    def cp(i, j):
        return pltpu.make_async_copy(
            mask_hbm.at[b, 0, pl.ds(base + _CH * i, _CH)], bufs[j], sems[j])

    cp(0, 0).start()
    cp(1, 1).start()

    def group(g, carry):
        for j in range(2):
            i = 2 * g + j
            cp(i, j).wait()

            def prbody(pr, cc):
                r0 = 16 * pr
                for v in range(_W // 16):
                    sl = pl.ds(16 * v, 16)
                    vals = [bufs[j][r0 + k, sl] for k in range(16)]
                    while len(vals) > 1:
                        vals = [vals[m] + vals[m + 1]
                                for m in range(0, len(vals), 2)]
                    rows_v[(_CH // 16) * i + pr, sl] = vals[0]
                return cc

            jax.lax.fori_loop(0, _CH // 16, prbody, 0)

            @pl.when(g < _NCH // 2 - 1)
            def _():
                cp(i + 2, j).start()
        return carry

    jax.lax.fori_loop(0, _NCH // 2, group, 0)

    def wpool(row, cc):
        ridx = jnp.full((16,), row, jnp.int32)
        for h2 in range(2):
            idx0 = 16 * lanes + 256 * h2
            vals = [plsc.load_gather(rows_v, [ridx, idx0 + l])
                    for l in range(16)]
            while len(vals) > 1:
                vals = [vals[m] + vals[m + 1] for m in range(0, len(vals), 2)]
            psum_v[row, pl.ds(16 * h2, 16)] = vals[0]
        return cc

    jax.lax.fori_loop(0, _QR // 16, wpool, 0)

    pltpu.sync_copy(psum_v, shared.at[s])
    plsc.subcore_barrier()

    @pl.when(q == 0)
    def _select():
        for m in range(4):
            pltpu.sync_copy(shared.at[s + m],
                            res_v.at[pl.ds(8 * m, _QR // 16)])
        pltpu.sync_copy(tbl_hbm.at[b], tbl_v)

        thresh = jnp.full((16,), float(_P * _P), jnp.float32)
        one = jnp.full((16,), 1, jnp.int32)
        zero = jnp.full((16,), 0, jnp.int32)

        def count(r, n):
            for p in range(2):
                v = res_v[r, pl.ds(p * 16, 16)]
                n = n + jnp.sum(jnp.where(v < thresh, one, zero))
            return n

        n = jax.lax.fori_loop(0, _HP, count, jnp.int32(0))

        jv = plsc.load_gather(tbl_v, [jnp.full((16,), n, jnp.int32)])
        t = jnp.max(jv) + 1

        def select(r, carry):
            run, flat = carry
            for p in range(2):
                v = res_v[r, pl.ds(p * 16, 16)]
                mi = jnp.where(v < thresh, one, zero)
                cs = jax.lax.cumsum(mi)
                cnt = jnp.sum(mi)
                sel = jnp.where(mi == one, cs, zero) == (t - run)
                pos = jnp.min(jnp.where(sel, lanes, 16))
                hit = (run < t) & (t <= run + cnt)
                flat = jnp.where(hit, r * 32 + p * 16 + pos, flat)
                run = run + cnt
            return run, flat

        _, flat = jax.lax.fori_loop(0, _HP, select,
                                    (jnp.int32(0), jnp.int32(0)))

        h = (flat >> 5) << 4
        w = (flat & 31) << 4
        out_v[...] = jnp.where(lanes == 0, h, jnp.where(lanes == 1, w, 0))

        @pl.when(c == 0)
        def _():
            pltpu.sync_copy(out_v, out0_hbm.at[s // 4])

        @pl.when(c == 1)
        def _():
            pltpu.sync_copy(out_v, out1_hbm.at[s // 4])


@jax.jit
def kernel(mask):
    sc0, sc1 = _sc_kernel(mask, _DRAW_TABLE)
    tch, tcw = pl.pallas_call(
        _tc_kernel,
        grid=(_BTC,),
        in_specs=[
            pl.BlockSpec((1, 1, _H, _W), lambda i: (i, 0, 0, 0)),
            pl.BlockSpec((1, 1, _TBL), lambda i: (i, 0, 0)),
        ],
        out_specs=[
            pl.BlockSpec((1, 1, 128), lambda i: (i, 0, 0)),
            pl.BlockSpec((1, 1, 128), lambda i: (i, 0, 0)),
        ],
        out_shape=[
            jax.ShapeDtypeStruct((_BTC, 1, 128), jnp.int32),
            jax.ShapeDtypeStruct((_BTC, 1, 128), jnp.int32),
        ],
    )(mask, _DRAW_TABLE_3D)
    sc = jnp.concatenate([sc0, sc1], axis=0)
    outh = jnp.concatenate([tch[:, 0, 0], sc[:, 0]])
    outw = jnp.concatenate([tcw[:, 0, 0], sc[:, 1]])
    return outh, outw
